# Initial kernel scaffold; baseline (speedup 1.0000x reference)
import jax
import jax.numpy as jnp
from jax.experimental import pallas as pl


def kernel(x, rbf, sbf, idx_kj, idx_ji, W_rbf, W_sbf, W_kj, b_kj, W_ji, b_ji, W_bil, W_lin, b_lin):
    raise NotImplementedError("write your pallas kernel here")


# speedup vs baseline: 1.5104x; 1.5104x over previous
import functools

import jax
import jax.numpy as jnp
from jax import lax
from jax.experimental import pallas as pl
from jax.experimental.pallas import tpu as pltpu
from jax.experimental.pallas import tpu_sc as plsc

E = 160000
T = 480000
H = 128
NB = 8
NR = 6
NS_SBF = 7 * 6

NC = 2
NSC = 16
NW = NC * NSC

f32 = jnp.float32
i32 = jnp.int32


def _swish(v):
    return v * jax.nn.sigmoid(v)


_EB = 2000


def _pro_body(x_ref, rbf_ref, wrbf_ref, wkj_ref, bkj_ref, wji_ref, bji_ref,
              xji_ref, xkj_ref):
    xv = x_ref[...]
    xji_ref[...] = _swish(
        jnp.dot(xv, wji_ref[...], preferred_element_type=f32) + bji_ref[...])
    rh = jnp.dot(rbf_ref[...], wrbf_ref[...], preferred_element_type=f32)
    xkj_ref[...] = _swish(
        jnp.dot(xv, wkj_ref[...], preferred_element_type=f32) + bkj_ref[...]) * rh


def _run_prologue(x, rbf, W_rbf, W_kj, b_kj, W_ji, b_ji):
    grid = (E // _EB,)
    row = lambda i: (i, 0)
    full = lambda i: (0, 0)
    return pl.pallas_call(
        _pro_body,
        grid=grid,
        in_specs=[
            pl.BlockSpec((_EB, H), row),
            pl.BlockSpec((_EB, NR), row),
            pl.BlockSpec((NR, H), full),
            pl.BlockSpec((H, H), full),
            pl.BlockSpec((1, H), full),
            pl.BlockSpec((H, H), full),
            pl.BlockSpec((1, H), full),
        ],
        out_specs=[pl.BlockSpec((_EB, H), row), pl.BlockSpec((_EB, H), row)],
        out_shape=[jax.ShapeDtypeStruct((E, H), f32),
                   jax.ShapeDtypeStruct((E, H), f32)],
    )(x, rbf, W_rbf, W_kj, b_kj.reshape(1, H), W_ji, b_ji.reshape(1, H))


_G_IT = 25
_G_ROWS = 600
_G_NS = 5
_G_SR = 120


def _gather_body(idx_hbm, src_hbm, out_hbm, idxv, rows, sem):
    c = lax.axis_index("c")
    s = lax.axis_index("s")
    wid = s * NC + c

    def chunk(i, carry):
        pltpu.sync_copy(idx_hbm.at[wid, i], idxv)
        for j in range(_G_NS):
            pltpu.async_copy(src_hbm.at[idxv.at[j]],
                             rows.at[pl.ds(j * _G_SR, _G_SR)], sem)
        for j in range(_G_NS):
            pltpu.make_async_copy(src_hbm.at[idxv.at[j]],
                                  rows.at[pl.ds(j * _G_SR, _G_SR)], sem).wait()
        off = (wid * _G_IT + i) * _G_ROWS
        pltpu.sync_copy(rows, out_hbm.at[pl.ds(off, _G_ROWS)])
        return carry

    lax.fori_loop(0, _G_IT, chunk, 0)


def _run_gather(idx_kj, x_kj):
    idx4 = idx_kj.reshape(NW, _G_IT, _G_NS, _G_SR).astype(i32)
    mesh = plsc.VectorSubcoreMesh(core_axis_name="c", subcore_axis_name="s")
    return pl.kernel(
        _gather_body,
        out_type=jax.ShapeDtypeStruct((T, H), f32),
        mesh=mesh,
        compiler_params=pltpu.CompilerParams(needs_layout_passes=False),
        scratch_types=[
            pltpu.VMEM((_G_NS, _G_SR), i32),
            pltpu.VMEM((_G_ROWS, H), f32),
            pltpu.SemaphoreType.DMA,
        ],
    )(idx4, x_kj)


_TB = 1280


def _bil_body(xg_ref, sbf_ref, wsbf_ref, w2_ref, out_ref):
    sh = jnp.dot(sbf_ref[...], wsbf_ref[...], preferred_element_type=f32)
    xg = xg_ref[...]
    acc = sh[:, 0:1] * jnp.dot(xg, w2_ref[0], preferred_element_type=f32)
    for j in range(1, NB):
        acc = acc + sh[:, j:j + 1] * jnp.dot(xg, w2_ref[j],
                                             preferred_element_type=f32)
    out_ref[...] = acc


def _run_bilinear(xg, sbf, W_sbf, W_bil):
    W2 = jnp.transpose(W_bil, (1, 2, 0))
    grid = (T // _TB,)
    row = lambda i: (i, 0)
    full2 = lambda i: (0, 0)
    full3 = lambda i: (0, 0, 0)
    return pl.pallas_call(
        _bil_body,
        grid=grid,
        in_specs=[
            pl.BlockSpec((_TB, H), row),
            pl.BlockSpec((_TB, NS_SBF), row),
            pl.BlockSpec((NS_SBF, NB), full2),
            pl.BlockSpec((NB, H, H), full3),
        ],
        out_specs=pl.BlockSpec((_TB, H), row),
        out_shape=jax.ShapeDtypeStruct((T, H), f32),
    )(xg, sbf, W_sbf, W2)


_S_CHUNKS_PER_CORE = 8
_S_CROWS = 10000
_S_FBLK = 200
_S_NFB = _S_CROWS // _S_FBLK
_S_PW = T // NSC
_S_BLK = 1200
_S_NBLK = _S_PW // _S_BLK
_S_NG = _S_BLK // 16
_S_FIRE = 128
_S_QCAP = 256
_S_DUMP = _S_CROWS


def _scatter_body(idx_hbm, xt_hbm, out_hbm, ib, tq, dq, dq2, rows, zbuf, acc,
                  sem):
    c = lax.axis_index("c")
    s = lax.axis_index("s")

    zv = jnp.zeros((16,), f32)

    def zinit(i, carry):
        r = i // 8
        col = (i % 8) * 16
        zbuf[r, pl.ds(col, 16)] = zv
        return carry

    lax.fori_loop(0, (200 * H) // 16, zinit, 0)

    dumpv = jnp.full((16,), _S_DUMP, i32)
    zidx = jnp.zeros((16,), i32)
    iota16 = lax.iota(i32, 16)

    def fire():
        cp = pltpu.async_copy(xt_hbm.at[tq.at[pl.ds(0, _S_FIRE)]], rows, sem)
        cp.wait()
        for kk in range(_S_FIRE // 16):
            dq2[pl.ds(kk * 16, 16)] = dq[pl.ds(kk * 16, 16)]
        pltpu.sync_copy(rows, acc.at[dq2], add=True)

    def shift_leftover():
        tl = tq[pl.ds(_S_FIRE, 16)]
        dl = dq[pl.ds(_S_FIRE, 16)]
        tq[pl.ds(0, 16)] = tl
        dq[pl.ds(0, 16)] = dl

    def one_chunk(k, carry0):
        chunk = c * _S_CHUNKS_PER_CORE + k
        lo = chunk * _S_CROWS

        def zrow(z, carry):
            bi = s + z * NSC

            def do():
                pltpu.sync_copy(zbuf, acc.at[pl.ds(bi * _S_FBLK, _S_FBLK)])

            pl.when(bi < _S_NFB)(do)
            return carry

        lax.fori_loop(0, (_S_NFB + NSC - 1) // NSC, zrow, 0)
        plsc.subcore_barrier()

        def blk_body(b, cnt):
            pltpu.sync_copy(idx_hbm.at[s, b], ib)

            def grp(g, cnt):
                v = ib[pl.ds(g * 16, 16)]
                m = (v >= lo) & (v < lo + _S_CROWS)
                t = (s * _S_NBLK + b) * _S_BLK + g * 16 + iota16
                d = v - lo
                mi = m.astype(i32)
                incl = plsc.cumsum(mi)
                pos = cnt + incl - mi
                plsc.store_scatter(tq, [pos], t, mask=m)
                plsc.store_scatter(dq, [pos], d, mask=m)
                cnt = cnt + jnp.sum(mi)

                def do_fire():
                    fire()
                    shift_leftover()

                pl.when(cnt >= _S_FIRE)(do_fire)
                return jnp.where(cnt >= _S_FIRE, cnt - _S_FIRE, cnt)

            return lax.fori_loop(0, _S_NG, grp, cnt)

        cnt = lax.fori_loop(0, _S_NBLK, blk_body, jnp.int32(0))

        def pad(j, carry):
            tq[pl.ds(cnt + j * 16, 16)] = zidx
            dq[pl.ds(cnt + j * 16, 16)] = dumpv
            return carry

        lax.fori_loop(0, 8, pad, 0)
        fire()
        plsc.subcore_barrier()

        def frow(z, carry):
            bi = s + z * NSC

            def do():
                r = bi * _S_FBLK
                pltpu.sync_copy(acc.at[pl.ds(r, _S_FBLK)],
                                out_hbm.at[pl.ds(lo + r, _S_FBLK)])

            pl.when(bi < _S_NFB)(do)
            return carry

        lax.fori_loop(0, (_S_NFB + NSC - 1) // NSC, frow, 0)
        return carry0

    lax.fori_loop(0, _S_CHUNKS_PER_CORE, one_chunk, 0)


def _run_scatter(idx_ji, xt):
    idx3 = idx_ji.reshape(NSC, _S_NBLK, _S_BLK).astype(i32)
    mesh = plsc.VectorSubcoreMesh(core_axis_name="c", subcore_axis_name="s")
    return pl.kernel(
        _scatter_body,
        out_type=jax.ShapeDtypeStruct((E, H), f32),
        mesh=mesh,
        compiler_params=pltpu.CompilerParams(needs_layout_passes=False),
        scratch_types=[
            pltpu.VMEM((_S_BLK,), i32),
            pltpu.VMEM((_S_QCAP,), i32),
            pltpu.VMEM((_S_QCAP,), i32),
            pltpu.VMEM((_S_FIRE,), i32),
            pltpu.VMEM((_S_FIRE, H), f32),
            pltpu.VMEM((200, H), f32),
            pltpu.VMEM_SHARED((_S_CROWS + 8, H), f32),
            pltpu.SemaphoreType.DMA,
        ],
    )(idx3, xt)


def _epi_body(xji_ref, agg_ref, wlin_ref, blin_ref, out_ref):
    hv = xji_ref[...] + agg_ref[...]
    out_ref[...] = _swish(
        jnp.dot(hv, wlin_ref[...], preferred_element_type=f32) + blin_ref[...])


def _run_epilogue(x_ji, agg, W_lin, b_lin):
    grid = (E // _EB,)
    row = lambda i: (i, 0)
    full = lambda i: (0, 0)
    return pl.pallas_call(
        _epi_body,
        grid=grid,
        in_specs=[
            pl.BlockSpec((_EB, H), row),
            pl.BlockSpec((_EB, H), row),
            pl.BlockSpec((H, H), full),
            pl.BlockSpec((1, H), full),
        ],
        out_specs=pl.BlockSpec((_EB, H), row),
        out_shape=jax.ShapeDtypeStruct((E, H), f32),
    )(x_ji, agg, W_lin, b_lin.reshape(1, H))


def kernel(x, rbf, sbf, idx_kj, idx_ji, W_rbf, W_sbf, W_kj, b_kj, W_ji, b_ji,
           W_bil, W_lin, b_lin):
    x_ji, x_kj = _run_prologue(x, rbf, W_rbf, W_kj, b_kj, W_ji, b_ji)
    xg = _run_gather(idx_kj, x_kj)
    xt = _run_bilinear(xg, sbf, W_sbf, W_bil)
    agg = _run_scatter(idx_ji, xt)
    return _run_epilogue(x_ji, agg, W_lin, b_lin)


# speedup vs baseline: 1.5125x; 1.0014x over previous
import functools

import jax
import jax.numpy as jnp
from jax import lax
from jax.experimental import pallas as pl
from jax.experimental.pallas import tpu as pltpu
from jax.experimental.pallas import tpu_sc as plsc

E = 160000
T = 480000
H = 128
NB = 8
NR = 6
NS_SBF = 7 * 6

NC = 2
NSC = 16
NW = NC * NSC

f32 = jnp.float32
i32 = jnp.int32


def _swish(v):
    return v * jax.nn.sigmoid(v)


_EB = 2000


def _pro_body(x_ref, rbf_ref, wrbf_ref, wkj_ref, bkj_ref, wji_ref, bji_ref,
              xji_ref, xkj_ref):
    xv = x_ref[...]
    xji_ref[...] = _swish(
        jnp.dot(xv, wji_ref[...], preferred_element_type=f32) + bji_ref[...])
    rh = jnp.dot(rbf_ref[...], wrbf_ref[...], preferred_element_type=f32)
    xkj_ref[...] = _swish(
        jnp.dot(xv, wkj_ref[...], preferred_element_type=f32) + bkj_ref[...]) * rh


def _run_prologue(x, rbf, W_rbf, W_kj, b_kj, W_ji, b_ji):
    grid = (E // _EB,)
    row = lambda i: (i, 0)
    full = lambda i: (0, 0)
    return pl.pallas_call(
        _pro_body,
        grid=grid,
        in_specs=[
            pl.BlockSpec((_EB, H), row),
            pl.BlockSpec((_EB, NR), row),
            pl.BlockSpec((NR, H), full),
            pl.BlockSpec((H, H), full),
            pl.BlockSpec((1, H), full),
            pl.BlockSpec((H, H), full),
            pl.BlockSpec((1, H), full),
        ],
        out_specs=[pl.BlockSpec((_EB, H), row), pl.BlockSpec((_EB, H), row)],
        out_shape=[jax.ShapeDtypeStruct((E, H), f32),
                   jax.ShapeDtypeStruct((E, H), f32)],
    )(x, rbf, W_rbf, W_kj, b_kj.reshape(1, H), W_ji, b_ji.reshape(1, H))


_G_IT = 25
_G_ROWS = 600
_G_NS = 5
_G_SR = 120


def _gather_body(idx_hbm, src_hbm, out_hbm, idxv, rows, sem):
    c = lax.axis_index("c")
    s = lax.axis_index("s")
    wid = s * NC + c

    def chunk(i, carry):
        pltpu.sync_copy(idx_hbm.at[wid, i], idxv)
        for j in range(_G_NS):
            pltpu.async_copy(src_hbm.at[idxv.at[j]],
                             rows.at[pl.ds(j * _G_SR, _G_SR)], sem)
        for j in range(_G_NS):
            pltpu.make_async_copy(src_hbm.at[idxv.at[j]],
                                  rows.at[pl.ds(j * _G_SR, _G_SR)], sem).wait()
        off = (wid * _G_IT + i) * _G_ROWS
        pltpu.sync_copy(rows, out_hbm.at[pl.ds(off, _G_ROWS)])
        return carry

    lax.fori_loop(0, _G_IT, chunk, 0)


def _run_gather(idx_kj, x_kj):
    idx4 = idx_kj.reshape(NW, _G_IT, _G_NS, _G_SR).astype(i32)
    mesh = plsc.VectorSubcoreMesh(core_axis_name="c", subcore_axis_name="s")
    return pl.kernel(
        _gather_body,
        out_type=jax.ShapeDtypeStruct((T, H), f32),
        mesh=mesh,
        compiler_params=pltpu.CompilerParams(needs_layout_passes=False),
        scratch_types=[
            pltpu.VMEM((_G_NS, _G_SR), i32),
            pltpu.VMEM((_G_ROWS, H), f32),
            pltpu.SemaphoreType.DMA,
        ],
    )(idx4, x_kj)


_TB = 1280


def _bil_body(xg_ref, sbf_ref, wsbf_ref, w2_ref, out_ref):
    sh = jnp.dot(sbf_ref[...], wsbf_ref[...], preferred_element_type=f32)
    xg = xg_ref[...].astype(jnp.bfloat16)
    acc = sh[:, 0:1] * jnp.dot(xg, w2_ref[0], preferred_element_type=f32)
    for j in range(1, NB):
        acc = acc + sh[:, j:j + 1] * jnp.dot(xg, w2_ref[j],
                                             preferred_element_type=f32)
    out_ref[...] = acc


def _run_bilinear(xg, sbf, W_sbf, W_bil):
    W2 = jnp.transpose(W_bil, (1, 2, 0)).astype(jnp.bfloat16)
    grid = (T // _TB,)
    row = lambda i: (i, 0)
    full2 = lambda i: (0, 0)
    full3 = lambda i: (0, 0, 0)
    return pl.pallas_call(
        _bil_body,
        grid=grid,
        in_specs=[
            pl.BlockSpec((_TB, H), row),
            pl.BlockSpec((_TB, NS_SBF), row),
            pl.BlockSpec((NS_SBF, NB), full2),
            pl.BlockSpec((NB, H, H), full3),
        ],
        out_specs=pl.BlockSpec((_TB, H), row),
        out_shape=jax.ShapeDtypeStruct((T, H), f32),
    )(xg, sbf, W_sbf, W2)


_S_CHUNKS_PER_CORE = 8
_S_CROWS = 10000
_S_FBLK = 200
_S_NFB = _S_CROWS // _S_FBLK
_S_PW = T // NSC
_S_BLK = 1200
_S_NBLK = _S_PW // _S_BLK
_S_NG = _S_BLK // 16
_S_FIRE = 128
_S_QCAP = 256
_S_DUMP = _S_CROWS


def _scatter_body(idx_hbm, xt_hbm, out_hbm, ib, tq, dq, dq2, rows, zbuf, acc,
                  sem):
    c = lax.axis_index("c")
    s = lax.axis_index("s")

    zv = jnp.zeros((16,), f32)

    def zinit(i, carry):
        r = i // 8
        col = (i % 8) * 16
        zbuf[r, pl.ds(col, 16)] = zv
        return carry

    lax.fori_loop(0, (200 * H) // 16, zinit, 0)

    dumpv = jnp.full((16,), _S_DUMP, i32)
    zidx = jnp.zeros((16,), i32)
    iota16 = lax.iota(i32, 16)

    def fire():
        cp = pltpu.async_copy(xt_hbm.at[tq.at[pl.ds(0, _S_FIRE)]], rows, sem)
        cp.wait()
        for kk in range(_S_FIRE // 16):
            dq2[pl.ds(kk * 16, 16)] = dq[pl.ds(kk * 16, 16)]
        pltpu.sync_copy(rows, acc.at[dq2], add=True)

    def shift_leftover():
        tl = tq[pl.ds(_S_FIRE, 16)]
        dl = dq[pl.ds(_S_FIRE, 16)]
        tq[pl.ds(0, 16)] = tl
        dq[pl.ds(0, 16)] = dl

    def one_chunk(k, carry0):
        chunk = c * _S_CHUNKS_PER_CORE + k
        lo = chunk * _S_CROWS

        def zrow(z, carry):
            bi = s + z * NSC

            def do():
                pltpu.sync_copy(zbuf, acc.at[pl.ds(bi * _S_FBLK, _S_FBLK)])

            pl.when(bi < _S_NFB)(do)
            return carry

        lax.fori_loop(0, (_S_NFB + NSC - 1) // NSC, zrow, 0)
        plsc.subcore_barrier()

        def blk_body(b, cnt):
            pltpu.sync_copy(idx_hbm.at[s, b], ib)

            def grp(g, cnt):
                v = ib[pl.ds(g * 16, 16)]
                m = (v >= lo) & (v < lo + _S_CROWS)
                t = (s * _S_NBLK + b) * _S_BLK + g * 16 + iota16
                d = v - lo
                mi = m.astype(i32)
                incl = plsc.cumsum(mi)
                pos = cnt + incl - mi
                plsc.store_scatter(tq, [pos], t, mask=m)
                plsc.store_scatter(dq, [pos], d, mask=m)
                cnt = cnt + jnp.sum(mi)

                def do_fire():
                    fire()
                    shift_leftover()

                pl.when(cnt >= _S_FIRE)(do_fire)
                return jnp.where(cnt >= _S_FIRE, cnt - _S_FIRE, cnt)

            return lax.fori_loop(0, _S_NG, grp, cnt)

        cnt = lax.fori_loop(0, _S_NBLK, blk_body, jnp.int32(0))

        def pad(j, carry):
            tq[pl.ds(cnt + j * 16, 16)] = zidx
            dq[pl.ds(cnt + j * 16, 16)] = dumpv
            return carry

        lax.fori_loop(0, 8, pad, 0)
        fire()
        plsc.subcore_barrier()

        def frow(z, carry):
            bi = s + z * NSC

            def do():
                r = bi * _S_FBLK
                pltpu.sync_copy(acc.at[pl.ds(r, _S_FBLK)],
                                out_hbm.at[pl.ds(lo + r, _S_FBLK)])

            pl.when(bi < _S_NFB)(do)
            return carry

        lax.fori_loop(0, (_S_NFB + NSC - 1) // NSC, frow, 0)
        return carry0

    lax.fori_loop(0, _S_CHUNKS_PER_CORE, one_chunk, 0)


def _run_scatter(idx_ji, xt):
    idx3 = idx_ji.reshape(NSC, _S_NBLK, _S_BLK).astype(i32)
    mesh = plsc.VectorSubcoreMesh(core_axis_name="c", subcore_axis_name="s")
    return pl.kernel(
        _scatter_body,
        out_type=jax.ShapeDtypeStruct((E, H), f32),
        mesh=mesh,
        compiler_params=pltpu.CompilerParams(needs_layout_passes=False),
        scratch_types=[
            pltpu.VMEM((_S_BLK,), i32),
            pltpu.VMEM((_S_QCAP,), i32),
            pltpu.VMEM((_S_QCAP,), i32),
            pltpu.VMEM((_S_FIRE,), i32),
            pltpu.VMEM((_S_FIRE, H), f32),
            pltpu.VMEM((200, H), f32),
            pltpu.VMEM_SHARED((_S_CROWS + 8, H), f32),
            pltpu.SemaphoreType.DMA,
        ],
    )(idx3, xt)


def _epi_body(xji_ref, agg_ref, wlin_ref, blin_ref, out_ref):
    hv = xji_ref[...] + agg_ref[...]
    out_ref[...] = _swish(
        jnp.dot(hv, wlin_ref[...], preferred_element_type=f32) + blin_ref[...])


def _run_epilogue(x_ji, agg, W_lin, b_lin):
    grid = (E // _EB,)
    row = lambda i: (i, 0)
    full = lambda i: (0, 0)
    return pl.pallas_call(
        _epi_body,
        grid=grid,
        in_specs=[
            pl.BlockSpec((_EB, H), row),
            pl.BlockSpec((_EB, H), row),
            pl.BlockSpec((H, H), full),
            pl.BlockSpec((1, H), full),
        ],
        out_specs=pl.BlockSpec((_EB, H), row),
        out_shape=jax.ShapeDtypeStruct((E, H), f32),
    )(x_ji, agg, W_lin, b_lin.reshape(1, H))


def kernel(x, rbf, sbf, idx_kj, idx_ji, W_rbf, W_sbf, W_kj, b_kj, W_ji, b_ji,
           W_bil, W_lin, b_lin):
    x_ji, x_kj = _run_prologue(x, rbf, W_rbf, W_kj, b_kj, W_ji, b_ji)
    xg = _run_gather(idx_kj, x_kj)
    xt = _run_bilinear(xg, sbf, W_sbf, W_bil)
    agg = _run_scatter(idx_ji, xt)
    return _run_epilogue(x_ji, agg, W_lin, b_lin)


# speedup vs baseline: 1.5627x; 1.0332x over previous
import functools

import jax
import jax.numpy as jnp
from jax import lax
from jax.experimental import pallas as pl
from jax.experimental.pallas import tpu as pltpu
from jax.experimental.pallas import tpu_sc as plsc

E = 160000
T = 480000
H = 128
NB = 8
NR = 6
NS_SBF = 7 * 6

NC = 2
NSC = 16
NW = NC * NSC

f32 = jnp.float32
i32 = jnp.int32


def _swish(v):
    return v * jax.nn.sigmoid(v)


_EB = 2000


def _pro_body(x_ref, rbf_ref, wrbf_ref, wkj_ref, bkj_ref, wji_ref, bji_ref,
              xji_ref, xkj_ref):
    xv = x_ref[...]
    xji_ref[...] = _swish(
        jnp.dot(xv, wji_ref[...], preferred_element_type=f32) + bji_ref[...])
    rh = jnp.dot(rbf_ref[...], wrbf_ref[...], preferred_element_type=f32)
    xkj_ref[...] = _swish(
        jnp.dot(xv, wkj_ref[...], preferred_element_type=f32) + bkj_ref[...]) * rh


def _run_prologue(x, rbf, W_rbf, W_kj, b_kj, W_ji, b_ji):
    grid = (E // _EB,)
    row = lambda i: (i, 0)
    full = lambda i: (0, 0)
    return pl.pallas_call(
        _pro_body,
        grid=grid,
        in_specs=[
            pl.BlockSpec((_EB, H), row),
            pl.BlockSpec((_EB, NR), row),
            pl.BlockSpec((NR, H), full),
            pl.BlockSpec((H, H), full),
            pl.BlockSpec((1, H), full),
            pl.BlockSpec((H, H), full),
            pl.BlockSpec((1, H), full),
        ],
        out_specs=[pl.BlockSpec((_EB, H), row), pl.BlockSpec((_EB, H), row)],
        out_shape=[jax.ShapeDtypeStruct((E, H), f32),
                   jax.ShapeDtypeStruct((E, H), f32)],
    )(x, rbf, W_rbf, W_kj, b_kj.reshape(1, H), W_ji, b_ji.reshape(1, H))


_G_IT = 25
_G_ROWS = 600
_G_NS = 5
_G_SR = 120


def _gather_body(idx_hbm, src_hbm, out_hbm, idxv, rows, sem):
    c = lax.axis_index("c")
    s = lax.axis_index("s")
    wid = s * NC + c

    def chunk(i, carry):
        pltpu.sync_copy(idx_hbm.at[wid, i], idxv)
        for j in range(_G_NS):
            pltpu.async_copy(src_hbm.at[idxv.at[j]],
                             rows.at[pl.ds(j * _G_SR, _G_SR)], sem)
        for j in range(_G_NS):
            pltpu.make_async_copy(src_hbm.at[idxv.at[j]],
                                  rows.at[pl.ds(j * _G_SR, _G_SR)], sem).wait()
        off = (wid * _G_IT + i) * _G_ROWS
        pltpu.sync_copy(rows, out_hbm.at[pl.ds(off, _G_ROWS)])
        return carry

    lax.fori_loop(0, _G_IT, chunk, 0)


def _run_gather(idx_kj, x_kj):
    idx4 = idx_kj.reshape(NW, _G_IT, _G_NS, _G_SR).astype(i32)
    mesh = plsc.VectorSubcoreMesh(core_axis_name="c", subcore_axis_name="s")
    return pl.kernel(
        _gather_body,
        out_type=jax.ShapeDtypeStruct((T, H), f32),
        mesh=mesh,
        compiler_params=pltpu.CompilerParams(needs_layout_passes=False),
        scratch_types=[
            pltpu.VMEM((_G_NS, _G_SR), i32),
            pltpu.VMEM((_G_ROWS, H), f32),
            pltpu.SemaphoreType.DMA,
        ],
    )(idx4, x_kj)


_TB = 1280


def _bil_body(xg_ref, sbf_ref, wsbf_ref, w2_ref, out_ref):
    sh = jnp.dot(sbf_ref[...], wsbf_ref[...], preferred_element_type=f32)
    xg = xg_ref[...].astype(jnp.bfloat16)
    acc = sh[:, 0:1] * jnp.dot(xg, w2_ref[0], preferred_element_type=f32)
    for j in range(1, NB):
        acc = acc + sh[:, j:j + 1] * jnp.dot(xg, w2_ref[j],
                                             preferred_element_type=f32)
    out_ref[...] = acc


def _run_bilinear(xg, sbf, W_sbf, W_bil):
    W2 = jnp.transpose(W_bil, (1, 2, 0)).astype(jnp.bfloat16)
    grid = (T // _TB,)
    row = lambda i: (i, 0)
    full2 = lambda i: (0, 0)
    full3 = lambda i: (0, 0, 0)
    return pl.pallas_call(
        _bil_body,
        grid=grid,
        in_specs=[
            pl.BlockSpec((_TB, H), row),
            pl.BlockSpec((_TB, NS_SBF), row),
            pl.BlockSpec((NS_SBF, NB), full2),
            pl.BlockSpec((NB, H, H), full3),
        ],
        out_specs=pl.BlockSpec((_TB, H), row),
        out_shape=jax.ShapeDtypeStruct((T, H), f32),
    )(xg, sbf, W_sbf, W2)


_S_CHUNKS_PER_CORE = 10
_S_CROWS = 8000
_S_FBLK = 200
_S_NFB = _S_CROWS // _S_FBLK
_S_PW = T // NSC
_S_BLK = 1200
_S_NBLK = _S_PW // _S_BLK
_S_NG = _S_BLK // 16
_S_FIRE = 128
_S_QCAP = 256
_S_DUMP = _S_CROWS


def _scatter_body(idx_hbm, xt_hbm, out_hbm, ib, tq, dq, dq2, rows, zbuf, acc,
                  sem, sem_a):
    c = lax.axis_index("c")
    s = lax.axis_index("s")

    zv = jnp.zeros((16,), f32)

    def zinit(i, carry):
        r = i // 8
        col = (i % 8) * 16
        zbuf[r, pl.ds(col, 16)] = zv
        return carry

    lax.fori_loop(0, (200 * H) // 16, zinit, 0)

    dumpv = jnp.full((16,), _S_DUMP, i32)
    zidx = jnp.zeros((16,), i32)
    iota16 = lax.iota(i32, 16)

    def wait_gather(p):
        pltpu.make_async_copy(xt_hbm.at[dq2.at[p]], rows.at[p], sem).wait()

    def issue_add(p):
        pltpu.async_copy(rows.at[p], acc.at[dq2.at[p + 2]], sem_a, add=True)

    def wait_add(p):
        pltpu.make_async_copy(rows.at[p], acc.at[dq2.at[p + 2]], sem_a).wait()

    def fire(nf):
        p = nf % 2

        def prev_add():
            wait_gather(1 - p)
            issue_add(1 - p)

        pl.when(nf >= 1)(prev_add)
        pl.when(nf >= 2)(lambda: wait_add(p))
        for kk in range(_S_FIRE // 16):
            dq2[p, pl.ds(kk * 16, 16)] = tq[pl.ds(kk * 16, 16)]
            dq2[p + 2, pl.ds(kk * 16, 16)] = dq[pl.ds(kk * 16, 16)]
        tl = tq[pl.ds(_S_FIRE, 16)]
        dl = dq[pl.ds(_S_FIRE, 16)]
        tq[pl.ds(0, 16)] = tl
        dq[pl.ds(0, 16)] = dl
        pltpu.async_copy(xt_hbm.at[dq2.at[p]], rows.at[p], sem)

    def drain(nf_last):
        p = nf_last % 2
        wait_gather(p)
        issue_add(p)
        pl.when(nf_last >= 1)(lambda: wait_add(1 - p))
        wait_add(p)

    def one_chunk(k, carry0):
        chunk = c * _S_CHUNKS_PER_CORE + k
        lo = chunk * _S_CROWS

        def zrow(z, carry):
            bi = s + z * NSC

            def do():
                pltpu.sync_copy(zbuf, acc.at[pl.ds(bi * _S_FBLK, _S_FBLK)])

            pl.when(bi < _S_NFB)(do)
            return carry

        lax.fori_loop(0, (_S_NFB + NSC - 1) // NSC, zrow, 0)
        plsc.subcore_barrier()

        def blk_body(b, carry):
            pltpu.sync_copy(idx_hbm.at[s, b], ib)

            def grp(g, carry):
                cnt, nf = carry
                v = ib[pl.ds(g * 16, 16)]
                m = (v >= lo) & (v < lo + _S_CROWS)
                t = (s * _S_NBLK + b) * _S_BLK + g * 16 + iota16
                d = v - lo
                mi = m.astype(i32)
                incl = plsc.cumsum(mi)
                pos = cnt + incl - mi
                plsc.store_scatter(tq, [pos], t, mask=m)
                plsc.store_scatter(dq, [pos], d, mask=m)
                cnt = cnt + jnp.sum(mi)
                full = cnt >= _S_FIRE
                pl.when(full)(lambda: fire(nf))
                return (jnp.where(full, cnt - _S_FIRE, cnt),
                        jnp.where(full, nf + 1, nf))

            return lax.fori_loop(0, _S_NG, grp, carry)

        cnt, nf = lax.fori_loop(0, _S_NBLK, blk_body,
                                (jnp.int32(0), jnp.int32(0)))

        def pad(j, carry):
            tq[pl.ds(cnt + j * 16, 16)] = zidx
            dq[pl.ds(cnt + j * 16, 16)] = dumpv
            return carry

        lax.fori_loop(0, 8, pad, 0)
        fire(nf)
        drain(nf)
        plsc.subcore_barrier()

        def frow(z, carry):
            bi = s + z * NSC

            def do():
                r = bi * _S_FBLK
                pltpu.sync_copy(acc.at[pl.ds(r, _S_FBLK)],
                                out_hbm.at[pl.ds(lo + r, _S_FBLK)])

            pl.when(bi < _S_NFB)(do)
            return carry

        lax.fori_loop(0, (_S_NFB + NSC - 1) // NSC, frow, 0)
        return carry0

    lax.fori_loop(0, _S_CHUNKS_PER_CORE, one_chunk, 0)


def _run_scatter(idx_ji, xt):
    idx3 = idx_ji.reshape(NSC, _S_NBLK, _S_BLK).astype(i32)
    mesh = plsc.VectorSubcoreMesh(core_axis_name="c", subcore_axis_name="s")
    return pl.kernel(
        _scatter_body,
        out_type=jax.ShapeDtypeStruct((E, H), f32),
        mesh=mesh,
        compiler_params=pltpu.CompilerParams(needs_layout_passes=False),
        scratch_types=[
            pltpu.VMEM((_S_BLK,), i32),
            pltpu.VMEM((_S_QCAP,), i32),
            pltpu.VMEM((_S_QCAP,), i32),
            pltpu.VMEM((4, _S_FIRE), i32),
            pltpu.VMEM((2, _S_FIRE, H), f32),
            pltpu.VMEM((200, H), f32),
            pltpu.VMEM_SHARED((_S_CROWS + 8, H), f32),
            pltpu.SemaphoreType.DMA,
            pltpu.SemaphoreType.DMA,
        ],
    )(idx3, xt)


def _epi_body(xji_ref, agg_ref, wlin_ref, blin_ref, out_ref):
    hv = xji_ref[...] + agg_ref[...]
    out_ref[...] = _swish(
        jnp.dot(hv, wlin_ref[...], preferred_element_type=f32) + blin_ref[...])


def _run_epilogue(x_ji, agg, W_lin, b_lin):
    grid = (E // _EB,)
    row = lambda i: (i, 0)
    full = lambda i: (0, 0)
    return pl.pallas_call(
        _epi_body,
        grid=grid,
        in_specs=[
            pl.BlockSpec((_EB, H), row),
            pl.BlockSpec((_EB, H), row),
            pl.BlockSpec((H, H), full),
            pl.BlockSpec((1, H), full),
        ],
        out_specs=pl.BlockSpec((_EB, H), row),
        out_shape=jax.ShapeDtypeStruct((E, H), f32),
    )(x_ji, agg, W_lin, b_lin.reshape(1, H))


def kernel(x, rbf, sbf, idx_kj, idx_ji, W_rbf, W_sbf, W_kj, b_kj, W_ji, b_ji,
           W_bil, W_lin, b_lin):
    x_ji, x_kj = _run_prologue(x, rbf, W_rbf, W_kj, b_kj, W_ji, b_ji)
    xg = _run_gather(idx_kj, x_kj)
    xt = _run_bilinear(xg, sbf, W_sbf, W_bil)
    agg = _run_scatter(idx_ji, xt)
    return _run_epilogue(x_ji, agg, W_lin, b_lin)


# speedup vs baseline: 1.6982x; 1.0867x over previous
import functools

import jax
import jax.numpy as jnp
from jax import lax
from jax.experimental import pallas as pl
from jax.experimental.pallas import tpu as pltpu
from jax.experimental.pallas import tpu_sc as plsc

E = 160000
T = 480000
H = 128
NB = 8
NR = 6
NS_SBF = 7 * 6

NC = 2
NSC = 16
NW = NC * NSC

f32 = jnp.float32
i32 = jnp.int32


def _swish(v):
    return v * jax.nn.sigmoid(v)


_EB = 2000


def _pro_body(x_ref, rbf_ref, wrbf_ref, wkj_ref, bkj_ref, wji_ref, bji_ref,
              xji_ref, xkj_ref):
    xv = x_ref[...]
    xji_ref[...] = _swish(
        jnp.dot(xv, wji_ref[...], preferred_element_type=f32) + bji_ref[...])
    rh = jnp.dot(rbf_ref[...], wrbf_ref[...], preferred_element_type=f32)
    xkj_ref[...] = _swish(
        jnp.dot(xv, wkj_ref[...], preferred_element_type=f32) + bkj_ref[...]) * rh


def _run_prologue(x, rbf, W_rbf, W_kj, b_kj, W_ji, b_ji):
    grid = (E // _EB,)
    row = lambda i: (i, 0)
    full = lambda i: (0, 0)
    return pl.pallas_call(
        _pro_body,
        grid=grid,
        in_specs=[
            pl.BlockSpec((_EB, H), row),
            pl.BlockSpec((_EB, NR), row),
            pl.BlockSpec((NR, H), full),
            pl.BlockSpec((H, H), full),
            pl.BlockSpec((1, H), full),
            pl.BlockSpec((H, H), full),
            pl.BlockSpec((1, H), full),
        ],
        out_specs=[pl.BlockSpec((_EB, H), row), pl.BlockSpec((_EB, H), row)],
        out_shape=[jax.ShapeDtypeStruct((E, H), f32),
                   jax.ShapeDtypeStruct((E, H), f32)],
    )(x, rbf, W_rbf, W_kj, b_kj.reshape(1, H), W_ji, b_ji.reshape(1, H))


_G_IT = 25
_G_ROWS = 600
_G_NS = 5
_G_SR = 120


def _gather_body(idx_hbm, src_hbm, out_hbm, idxv, rows, sem):
    c = lax.axis_index("c")
    s = lax.axis_index("s")
    wid = s * NC + c

    def chunk(i, carry):
        pltpu.sync_copy(idx_hbm.at[wid, i], idxv)
        for j in range(_G_NS):
            pltpu.async_copy(src_hbm.at[idxv.at[j]],
                             rows.at[pl.ds(j * _G_SR, _G_SR)], sem)
        for j in range(_G_NS):
            pltpu.make_async_copy(src_hbm.at[idxv.at[j]],
                                  rows.at[pl.ds(j * _G_SR, _G_SR)], sem).wait()
        off = (wid * _G_IT + i) * _G_ROWS
        pltpu.sync_copy(rows, out_hbm.at[pl.ds(off, _G_ROWS)])
        return carry

    lax.fori_loop(0, _G_IT, chunk, 0)


def _run_gather(idx_kj, x_kj):
    idx4 = idx_kj.reshape(NW, _G_IT, _G_NS, _G_SR).astype(i32)
    mesh = plsc.VectorSubcoreMesh(core_axis_name="c", subcore_axis_name="s")
    return pl.kernel(
        _gather_body,
        out_type=jax.ShapeDtypeStruct((T, H), f32),
        mesh=mesh,
        compiler_params=pltpu.CompilerParams(needs_layout_passes=False),
        scratch_types=[
            pltpu.VMEM((_G_NS, _G_SR), i32),
            pltpu.VMEM((_G_ROWS, H), f32),
            pltpu.SemaphoreType.DMA,
        ],
    )(idx4, x_kj)


_TB = 1280


def _bil_body(xg_ref, sbf_ref, wsbf_ref, w2_ref, out_ref):
    sh = jnp.dot(sbf_ref[...], wsbf_ref[...], preferred_element_type=f32)
    xg = xg_ref[...].astype(jnp.bfloat16)
    acc = sh[:, 0:1] * jnp.dot(xg, w2_ref[0], preferred_element_type=f32)
    for j in range(1, NB):
        acc = acc + sh[:, j:j + 1] * jnp.dot(xg, w2_ref[j],
                                             preferred_element_type=f32)
    out_ref[...] = acc


def _run_bilinear(xg, sbf, W_sbf, W_bil):
    W2 = jnp.transpose(W_bil, (1, 2, 0)).astype(jnp.bfloat16)
    grid = (T // _TB,)
    row = lambda i: (i, 0)
    full2 = lambda i: (0, 0)
    full3 = lambda i: (0, 0, 0)
    return pl.pallas_call(
        _bil_body,
        grid=grid,
        in_specs=[
            pl.BlockSpec((_TB, H), row),
            pl.BlockSpec((_TB, NS_SBF), row),
            pl.BlockSpec((NS_SBF, NB), full2),
            pl.BlockSpec((NB, H, H), full3),
        ],
        out_specs=pl.BlockSpec((_TB, H), row),
        out_shape=jax.ShapeDtypeStruct((T, H), f32),
    )(xg, sbf, W_sbf, W2)


_S_CHUNKS_PER_CORE = 8
_S_CROWS = 10000
_S_FBLK = 80
_S_NFB = _S_CROWS // _S_FBLK
_S_PW = T // NSC
_S_BLK = 1200
_S_NBLK = _S_PW // _S_BLK
_S_NG = _S_BLK // 16
_S_FIRE = 128
_S_QCAP = 256
_S_DUMP = _S_CROWS


def _scatter_body(idx_hbm, xt_hbm, out_hbm, ib, tq, dq, dq2, rows, zbuf, acc,
                  sem, sem_a):
    c = lax.axis_index("c")
    s = lax.axis_index("s")

    zv = jnp.zeros((16,), f32)

    def zinit(i, carry):
        r = i // 8
        col = (i % 8) * 16
        zbuf[r, pl.ds(col, 16)] = zv
        return carry

    lax.fori_loop(0, (_S_FBLK * H) // 16, zinit, 0)

    dumpv = jnp.full((16,), _S_DUMP, i32)
    zidx = jnp.zeros((16,), i32)
    iota16 = lax.iota(i32, 16)

    def wait_gather(p):
        pltpu.make_async_copy(xt_hbm.at[dq2.at[p]], rows.at[p], sem).wait()

    def issue_add(p):
        pltpu.async_copy(rows.at[p], acc.at[dq2.at[p + 2]], sem_a, add=True)

    def wait_add(p):
        pltpu.make_async_copy(rows.at[p], acc.at[dq2.at[p + 2]], sem_a).wait()

    def fire(nf):
        p = nf % 2

        def prev_add():
            wait_gather(1 - p)
            issue_add(1 - p)

        pl.when(nf >= 1)(prev_add)
        pl.when(nf >= 2)(lambda: wait_add(p))
        for kk in range(_S_FIRE // 16):
            dq2[p, pl.ds(kk * 16, 16)] = tq[pl.ds(kk * 16, 16)]
            dq2[p + 2, pl.ds(kk * 16, 16)] = dq[pl.ds(kk * 16, 16)]
        tl = tq[pl.ds(_S_FIRE, 16)]
        dl = dq[pl.ds(_S_FIRE, 16)]
        tq[pl.ds(0, 16)] = tl
        dq[pl.ds(0, 16)] = dl
        pltpu.async_copy(xt_hbm.at[dq2.at[p]], rows.at[p], sem)

    def drain(nf_last):
        p = nf_last % 2
        wait_gather(p)
        issue_add(p)
        pl.when(nf_last >= 1)(lambda: wait_add(1 - p))
        wait_add(p)

    def one_chunk(k, carry0):
        chunk = c * _S_CHUNKS_PER_CORE + k
        lo = chunk * _S_CROWS

        def zrow(z, carry):
            bi = s + z * NSC

            def do():
                pltpu.sync_copy(zbuf, acc.at[pl.ds(bi * _S_FBLK, _S_FBLK)])

            pl.when(bi < _S_NFB)(do)
            return carry

        lax.fori_loop(0, (_S_NFB + NSC - 1) // NSC, zrow, 0)
        plsc.subcore_barrier()

        def blk_body(b, carry):
            pltpu.sync_copy(idx_hbm.at[s, b], ib)

            def grp(g, carry):
                cnt, nf = carry
                v = ib[pl.ds(g * 16, 16)]
                m = (v >= lo) & (v < lo + _S_CROWS)
                t = (s * _S_NBLK + b) * _S_BLK + g * 16 + iota16
                d = v - lo
                mi = m.astype(i32)
                n = jnp.sum(mi)

                def append():
                    incl = plsc.cumsum(mi)
                    pos = cnt + incl - mi
                    plsc.store_scatter(tq, [pos], t, mask=m)
                    plsc.store_scatter(dq, [pos], d, mask=m)

                pl.when(n > 0)(append)
                cnt = cnt + n
                full = cnt >= _S_FIRE
                pl.when(full)(lambda: fire(nf))
                return (jnp.where(full, cnt - _S_FIRE, cnt),
                        jnp.where(full, nf + 1, nf))

            return lax.fori_loop(0, _S_NG, grp, carry)

        cnt, nf = lax.fori_loop(0, _S_NBLK, blk_body,
                                (jnp.int32(0), jnp.int32(0)))

        def pad(j, carry):
            tq[pl.ds(cnt + j * 16, 16)] = zidx
            dq[pl.ds(cnt + j * 16, 16)] = dumpv
            return carry

        lax.fori_loop(0, 8, pad, 0)
        fire(nf)
        drain(nf)
        plsc.subcore_barrier()

        def frow(z, carry):
            bi = s + z * NSC

            def do():
                r = bi * _S_FBLK
                pltpu.sync_copy(acc.at[pl.ds(r, _S_FBLK)],
                                out_hbm.at[pl.ds(lo + r, _S_FBLK)])

            pl.when(bi < _S_NFB)(do)
            return carry

        lax.fori_loop(0, (_S_NFB + NSC - 1) // NSC, frow, 0)
        return carry0

    lax.fori_loop(0, _S_CHUNKS_PER_CORE, one_chunk, 0)


def _run_scatter(idx_ji, xt):
    idx3 = idx_ji.reshape(NSC, _S_NBLK, _S_BLK).astype(i32)
    mesh = plsc.VectorSubcoreMesh(core_axis_name="c", subcore_axis_name="s")
    return pl.kernel(
        _scatter_body,
        out_type=jax.ShapeDtypeStruct((E, H), f32),
        mesh=mesh,
        compiler_params=pltpu.CompilerParams(needs_layout_passes=False),
        scratch_types=[
            pltpu.VMEM((_S_BLK,), i32),
            pltpu.VMEM((_S_QCAP,), i32),
            pltpu.VMEM((_S_QCAP,), i32),
            pltpu.VMEM((4, _S_FIRE), i32),
            pltpu.VMEM((2, _S_FIRE, H), f32),
            pltpu.VMEM((_S_FBLK, H), f32),
            pltpu.VMEM_SHARED((_S_CROWS + 8, H), f32),
            pltpu.SemaphoreType.DMA,
            pltpu.SemaphoreType.DMA,
        ],
    )(idx3, xt)


def _epi_body(xji_ref, agg_ref, wlin_ref, blin_ref, out_ref):
    hv = xji_ref[...] + agg_ref[...]
    out_ref[...] = _swish(
        jnp.dot(hv, wlin_ref[...], preferred_element_type=f32) + blin_ref[...])


def _run_epilogue(x_ji, agg, W_lin, b_lin):
    grid = (E // _EB,)
    row = lambda i: (i, 0)
    full = lambda i: (0, 0)
    return pl.pallas_call(
        _epi_body,
        grid=grid,
        in_specs=[
            pl.BlockSpec((_EB, H), row),
            pl.BlockSpec((_EB, H), row),
            pl.BlockSpec((H, H), full),
            pl.BlockSpec((1, H), full),
        ],
        out_specs=pl.BlockSpec((_EB, H), row),
        out_shape=jax.ShapeDtypeStruct((E, H), f32),
    )(x_ji, agg, W_lin, b_lin.reshape(1, H))


def kernel(x, rbf, sbf, idx_kj, idx_ji, W_rbf, W_sbf, W_kj, b_kj, W_ji, b_ji,
           W_bil, W_lin, b_lin):
    x_ji, x_kj = _run_prologue(x, rbf, W_rbf, W_kj, b_kj, W_ji, b_ji)
    xg = _run_gather(idx_kj, x_kj)
    xt = _run_bilinear(xg, sbf, W_sbf, W_bil)
    agg = _run_scatter(idx_ji, xt)
    return _run_epilogue(x_ji, agg, W_lin, b_lin)


# speedup vs baseline: 1.7304x; 1.0190x over previous
import functools

import jax
import jax.numpy as jnp
from jax import lax
from jax.experimental import pallas as pl
from jax.experimental.pallas import tpu as pltpu
from jax.experimental.pallas import tpu_sc as plsc

E = 160000
T = 480000
H = 128
NB = 8
NR = 6
NS_SBF = 7 * 6

NC = 2
NSC = 16
NW = NC * NSC

f32 = jnp.float32
i32 = jnp.int32


def _swish(v):
    return v * jax.nn.sigmoid(v)


_EB = 2000


def _pro_body(x_ref, rbf_ref, wrbf_ref, wkj_ref, bkj_ref, wji_ref, bji_ref,
              xji_ref, xkj_ref):
    xv = x_ref[...]
    xji_ref[...] = _swish(
        jnp.dot(xv, wji_ref[...], preferred_element_type=f32) + bji_ref[...])
    rh = jnp.dot(rbf_ref[...], wrbf_ref[...], preferred_element_type=f32)
    xkj_ref[...] = _swish(
        jnp.dot(xv, wkj_ref[...], preferred_element_type=f32) + bkj_ref[...]) * rh


def _run_prologue(x, rbf, W_rbf, W_kj, b_kj, W_ji, b_ji):
    grid = (E // _EB,)
    row = lambda i: (i, 0)
    full = lambda i: (0, 0)
    return pl.pallas_call(
        _pro_body,
        grid=grid,
        in_specs=[
            pl.BlockSpec((_EB, H), row),
            pl.BlockSpec((_EB, NR), row),
            pl.BlockSpec((NR, H), full),
            pl.BlockSpec((H, H), full),
            pl.BlockSpec((1, H), full),
            pl.BlockSpec((H, H), full),
            pl.BlockSpec((1, H), full),
        ],
        out_specs=[pl.BlockSpec((_EB, H), row), pl.BlockSpec((_EB, H), row)],
        out_shape=[jax.ShapeDtypeStruct((E, H), f32),
                   jax.ShapeDtypeStruct((E, H), f32)],
    )(x, rbf, W_rbf, W_kj, b_kj.reshape(1, H), W_ji, b_ji.reshape(1, H))


_G_IT = 25
_G_ROWS = 600
_G_NS = 5
_G_SR = 120


def _gather_body(idx_hbm, src_hbm, out_hbm, idxv, rows, sem):
    c = lax.axis_index("c")
    s = lax.axis_index("s")
    wid = s * NC + c

    def chunk(i, carry):
        pltpu.sync_copy(idx_hbm.at[wid, i], idxv)
        for j in range(_G_NS):
            pltpu.async_copy(src_hbm.at[idxv.at[j]],
                             rows.at[pl.ds(j * _G_SR, _G_SR)], sem)
        for j in range(_G_NS):
            pltpu.make_async_copy(src_hbm.at[idxv.at[j]],
                                  rows.at[pl.ds(j * _G_SR, _G_SR)], sem).wait()
        off = (wid * _G_IT + i) * _G_ROWS
        pltpu.sync_copy(rows, out_hbm.at[pl.ds(off, _G_ROWS)])
        return carry

    lax.fori_loop(0, _G_IT, chunk, 0)


def _run_gather(idx_kj, x_kj):
    idx4 = idx_kj.reshape(NW, _G_IT, _G_NS, _G_SR).astype(i32)
    mesh = plsc.VectorSubcoreMesh(core_axis_name="c", subcore_axis_name="s")
    return pl.kernel(
        _gather_body,
        out_type=jax.ShapeDtypeStruct((T, H), f32),
        mesh=mesh,
        compiler_params=pltpu.CompilerParams(needs_layout_passes=False),
        scratch_types=[
            pltpu.VMEM((_G_NS, _G_SR), i32),
            pltpu.VMEM((_G_ROWS, H), f32),
            pltpu.SemaphoreType.DMA,
        ],
    )(idx4, x_kj)


_TB = 1280


def _bil_body(xg_ref, sbf_ref, wsbf_ref, w2_ref, out_ref):
    sh = jnp.dot(sbf_ref[...], wsbf_ref[...], preferred_element_type=f32)
    xg = xg_ref[...].astype(jnp.bfloat16)
    z = jnp.dot(xg, w2_ref[...], preferred_element_type=f32)
    acc = sh[:, 0:1] * z[:, 0:H]
    for j in range(1, NB):
        acc = acc + sh[:, j:j + 1] * z[:, j * H:(j + 1) * H]
    out_ref[...] = acc


def _run_bilinear(xg, sbf, W_sbf, W_bil):
    Wcat = jnp.transpose(W_bil, (2, 1, 0)).reshape(H, NB * H).astype(jnp.bfloat16)
    grid = (T // _TB,)
    row = lambda i: (i, 0)
    full2 = lambda i: (0, 0)
    return pl.pallas_call(
        _bil_body,
        grid=grid,
        in_specs=[
            pl.BlockSpec((_TB, H), row),
            pl.BlockSpec((_TB, NS_SBF), row),
            pl.BlockSpec((NS_SBF, NB), full2),
            pl.BlockSpec((H, NB * H), full2),
        ],
        out_specs=pl.BlockSpec((_TB, H), row),
        out_shape=jax.ShapeDtypeStruct((T, H), f32),
    )(xg, sbf, W_sbf, Wcat)


_S_CHUNKS_PER_CORE = 8
_S_CROWS = 10000
_S_FBLK = 80
_S_NFB = _S_CROWS // _S_FBLK
_S_PW = T // NSC
_S_BLK = 1200
_S_NBLK = _S_PW // _S_BLK
_S_NG = _S_BLK // 16
_S_FIRE = 128
_S_QCAP = 256
_S_DUMP = _S_CROWS


def _scatter_body(idx_hbm, xt_hbm, out_hbm, ib, tq, dq, dq2, rows, zbuf, acc,
                  sem, sem_a):
    c = lax.axis_index("c")
    s = lax.axis_index("s")

    zv = jnp.zeros((16,), f32)

    def zinit(i, carry):
        r = i // 8
        col = (i % 8) * 16
        zbuf[r, pl.ds(col, 16)] = zv
        return carry

    lax.fori_loop(0, (_S_FBLK * H) // 16, zinit, 0)

    dumpv = jnp.full((16,), _S_DUMP, i32)
    zidx = jnp.zeros((16,), i32)
    iota16 = lax.iota(i32, 16)

    def wait_gather(p):
        pltpu.make_async_copy(xt_hbm.at[dq2.at[p]], rows.at[p], sem).wait()

    def issue_add(p):
        pltpu.async_copy(rows.at[p], acc.at[dq2.at[p + 2]], sem_a, add=True)

    def wait_add(p):
        pltpu.make_async_copy(rows.at[p], acc.at[dq2.at[p + 2]], sem_a).wait()

    def fire(nf):
        p = nf % 2

        def prev_add():
            wait_gather(1 - p)
            issue_add(1 - p)

        pl.when(nf >= 1)(prev_add)
        pl.when(nf >= 2)(lambda: wait_add(p))
        for kk in range(_S_FIRE // 16):
            dq2[p, pl.ds(kk * 16, 16)] = tq[pl.ds(kk * 16, 16)]
            dq2[p + 2, pl.ds(kk * 16, 16)] = dq[pl.ds(kk * 16, 16)]
        tl = tq[pl.ds(_S_FIRE, 16)]
        dl = dq[pl.ds(_S_FIRE, 16)]
        tq[pl.ds(0, 16)] = tl
        dq[pl.ds(0, 16)] = dl
        pltpu.async_copy(xt_hbm.at[dq2.at[p]], rows.at[p], sem)

    def drain(nf_last):
        p = nf_last % 2
        wait_gather(p)
        issue_add(p)
        pl.when(nf_last >= 1)(lambda: wait_add(1 - p))
        wait_add(p)

    def one_chunk(k, carry0):
        chunk = c * _S_CHUNKS_PER_CORE + k
        lo = chunk * _S_CROWS

        def zrow(z, carry):
            bi = s + z * NSC

            def do():
                pltpu.sync_copy(zbuf, acc.at[pl.ds(bi * _S_FBLK, _S_FBLK)])

            pl.when(bi < _S_NFB)(do)
            return carry

        lax.fori_loop(0, (_S_NFB + NSC - 1) // NSC, zrow, 0)
        plsc.subcore_barrier()

        def blk_body(b, carry):
            pltpu.sync_copy(idx_hbm.at[s, b], ib)

            def grp(g, carry):
                cnt, nf = carry
                v = ib[pl.ds(g * 16, 16)]
                m = (v >= lo) & (v < lo + _S_CROWS)
                t = (s * _S_NBLK + b) * _S_BLK + g * 16 + iota16
                d = v - lo
                mi = m.astype(i32)
                n = jnp.sum(mi)

                def append():
                    incl = plsc.cumsum(mi)
                    pos = cnt + incl - mi
                    plsc.store_scatter(tq, [pos], t, mask=m)
                    plsc.store_scatter(dq, [pos], d, mask=m)

                pl.when(n > 0)(append)
                cnt = cnt + n
                full = cnt >= _S_FIRE
                pl.when(full)(lambda: fire(nf))
                return (jnp.where(full, cnt - _S_FIRE, cnt),
                        jnp.where(full, nf + 1, nf))

            return lax.fori_loop(0, _S_NG, grp, carry)

        cnt, nf = lax.fori_loop(0, _S_NBLK, blk_body,
                                (jnp.int32(0), jnp.int32(0)))

        def pad(j, carry):
            tq[pl.ds(cnt + j * 16, 16)] = zidx
            dq[pl.ds(cnt + j * 16, 16)] = dumpv
            return carry

        lax.fori_loop(0, 8, pad, 0)
        fire(nf)
        drain(nf)
        plsc.subcore_barrier()

        def frow(z, carry):
            bi = s + z * NSC

            def do():
                r = bi * _S_FBLK
                pltpu.sync_copy(acc.at[pl.ds(r, _S_FBLK)],
                                out_hbm.at[pl.ds(lo + r, _S_FBLK)])

            pl.when(bi < _S_NFB)(do)
            return carry

        lax.fori_loop(0, (_S_NFB + NSC - 1) // NSC, frow, 0)
        return carry0

    lax.fori_loop(0, _S_CHUNKS_PER_CORE, one_chunk, 0)


def _run_scatter(idx_ji, xt):
    idx3 = idx_ji.reshape(NSC, _S_NBLK, _S_BLK).astype(i32)
    mesh = plsc.VectorSubcoreMesh(core_axis_name="c", subcore_axis_name="s")
    return pl.kernel(
        _scatter_body,
        out_type=jax.ShapeDtypeStruct((E, H), f32),
        mesh=mesh,
        compiler_params=pltpu.CompilerParams(needs_layout_passes=False),
        scratch_types=[
            pltpu.VMEM((_S_BLK,), i32),
            pltpu.VMEM((_S_QCAP,), i32),
            pltpu.VMEM((_S_QCAP,), i32),
            pltpu.VMEM((4, _S_FIRE), i32),
            pltpu.VMEM((2, _S_FIRE, H), f32),
            pltpu.VMEM((_S_FBLK, H), f32),
            pltpu.VMEM_SHARED((_S_CROWS + 8, H), f32),
            pltpu.SemaphoreType.DMA,
            pltpu.SemaphoreType.DMA,
        ],
    )(idx3, xt)


def _epi_body(xji_ref, agg_ref, wlin_ref, blin_ref, out_ref):
    hv = xji_ref[...] + agg_ref[...]
    out_ref[...] = _swish(
        jnp.dot(hv, wlin_ref[...], preferred_element_type=f32) + blin_ref[...])


def _run_epilogue(x_ji, agg, W_lin, b_lin):
    grid = (E // _EB,)
    row = lambda i: (i, 0)
    full = lambda i: (0, 0)
    return pl.pallas_call(
        _epi_body,
        grid=grid,
        in_specs=[
            pl.BlockSpec((_EB, H), row),
            pl.BlockSpec((_EB, H), row),
            pl.BlockSpec((H, H), full),
            pl.BlockSpec((1, H), full),
        ],
        out_specs=pl.BlockSpec((_EB, H), row),
        out_shape=jax.ShapeDtypeStruct((E, H), f32),
    )(x_ji, agg, W_lin, b_lin.reshape(1, H))


def kernel(x, rbf, sbf, idx_kj, idx_ji, W_rbf, W_sbf, W_kj, b_kj, W_ji, b_ji,
           W_bil, W_lin, b_lin):
    x_ji, x_kj = _run_prologue(x, rbf, W_rbf, W_kj, b_kj, W_ji, b_ji)
    xg = _run_gather(idx_kj, x_kj)
    xt = _run_bilinear(xg, sbf, W_sbf, W_bil)
    agg = _run_scatter(idx_ji, xt)
    return _run_epilogue(x_ji, agg, W_lin, b_lin)
